# baseline (speedup 1.0000x reference)
import functools
import numpy as np
import jax
import jax.numpy as jnp
from jax import lax
from jax.experimental import pallas as pl
from jax.experimental.pallas import tpu as pltpu
from jax.experimental.pallas import tpu_sc as plsc

_N_PEGS = 32
_N_POS = _N_PEGS + 1
_N_ACT = _N_POS * 4

_N_ENVS = 65536
_NW = 32
_E_PER_W = _N_ENVS // _NW
_C = 256
_NCHUNK = _E_PER_W // _C
_NGROUP = _C // 16


def _build_tables():
    cells = [(i, j) for i in range(7) for j in range(7) if (2 <= i <= 4) or (2 <= j <= 4)]
    center = (3, 3)
    grid = [center] + [c for c in cells if c != center]
    moves = np.array([(-1, 0), (1, 0), (0, -1), (0, 1)], dtype=np.int64)
    grid_arr = np.array(grid, dtype=np.int64)
    pos_to_idx = {tuple(p): k for k, p in enumerate(grid)}
    action_pos_ids = np.repeat(np.arange(_N_POS), 4)
    action_move_ids = np.tile(np.arange(4), _N_POS)
    action_moves = moves[action_move_ids]
    action_positions = grid_arr[action_pos_ids]
    mid_pos = action_positions + action_moves
    tgt_pos = action_positions + 2 * action_moves
    mid_idx = np.zeros(_N_ACT, dtype=np.int64)
    tgt_idx = np.zeros(_N_ACT, dtype=np.int64)
    oob = np.zeros(_N_ACT, dtype=bool)
    for a in range(_N_ACT):
        m = tuple(mid_pos[a]); t = tuple(tgt_pos[a])
        if (m in pos_to_idx) and (t in pos_to_idx):
            mid_idx[a] = pos_to_idx[m]; tgt_idx[a] = pos_to_idx[t]
        else:
            oob[a] = True
            mid_idx[a] = pos_to_idx.get(m, 0)
            tgt_idx[a] = pos_to_idx.get(t, 0)
    return action_pos_ids, mid_idx, tgt_idx, oob, pos_to_idx


_APOS_PY, _AMID_PY, _ATGT_PY, _OOB_PY, _POS_TO_IDX = _build_tables()

_AMID_PAD = jnp.asarray(np.pad(_AMID_PY, (0, 12)).astype(np.int32))
_ATGT_PAD = jnp.asarray(np.pad(_ATGT_PY, (0, 12)).astype(np.int32))


def _sc_body(pegs_hbm, act_hbm, npegs_hbm, done_hbm, trew_hbm,
             amid_hbm, atgt_hbm,
             states_hbm, rew_hbm, ndone_hbm, npo_hbm, trewo_hbm,
             pegs_v, states_v, act_v, np_v, done_v, trew_v,
             rew_v, ndone_v, npo_v, trewo_v, amid_v, atgt_v):
    c = lax.axis_index("c")
    s = lax.axis_index("s")
    wid = s * 2 + c
    base_w = wid * _E_PER_W

    pltpu.sync_copy(amid_hbm, amid_v)
    pltpu.sync_copy(atgt_hbm, atgt_v)

    def chunk_body(ci, carry):
        base = base_w + ci * _C
        pltpu.sync_copy(pegs_hbm.at[pl.ds(base, _C)], pegs_v)
        pltpu.sync_copy(act_hbm.at[pl.ds(base, _C)], act_v)
        pltpu.sync_copy(npegs_hbm.at[pl.ds(base, _C)], np_v)
        pltpu.sync_copy(done_hbm.at[pl.ds(base, _C)], done_v)
        pltpu.sync_copy(trew_hbm.at[pl.ds(base, _C)], trew_v)

        def group_body(g, gcarry):
            e0 = g * 16
            rows = e0 + lax.iota(jnp.int32, 16)
            a = act_v[pl.ds(e0, 16)]
            pos = lax.shift_right_logical(a, 2)
            mid = plsc.load_gather(amid_v, [a])
            tgt = plsc.load_gather(atgt_v, [a])
            zeros = jnp.zeros((16,), jnp.float32)
            ones = jnp.ones((16,), jnp.float32)
            plsc.store_scatter(pegs_v, [rows, pos], zeros)
            plsc.store_scatter(pegs_v, [rows, mid], zeros)
            plsc.store_scatter(pegs_v, [rows, tgt], ones)

            v = [plsc.load_gather(pegs_v, [rows, jnp.full((16,), p, jnp.int32)])
                 for p in range(_N_POS)]

            ne0 = [None] * _N_POS
            gt0 = [None] * _N_POS
            ez = [None] * _N_POS
            feas = None
            for ai in range(_N_ACT):
                if _OOB_PY[ai]:
                    continue
                pa = int(_APOS_PY[ai]); ma = int(_AMID_PY[ai]); ta = int(_ATGT_PY[ai])
                if ne0[pa] is None:
                    ne0[pa] = v[pa] != 0.0
                if gt0[ma] is None:
                    gt0[ma] = v[ma] > 0.0
                if ez[ta] is None:
                    ez[ta] = v[ta] == 0.0
                t = ne0[pa] & gt0[ma] & ez[ta]
                feas = t if feas is None else (feas | t)

            done_g = done_v[pl.ds(e0, 16)]
            feas = feas & (done_g == 0)
            n_new = np_v[pl.ds(e0, 16)] - 1
            done_win = n_new == 1
            rew = jnp.where(done_win,
                            jnp.full((16,), 1.0, jnp.float32),
                            jnp.full((16,), 1.0 / (_N_PEGS - 1), jnp.float32))
            new_done = (done_win | (~feas)).astype(jnp.int32)
            nf = n_new.astype(jnp.float32)
            r1 = (nf - 1.0) * (1.0 / (_N_PEGS - 1))
            r2 = (_N_PEGS - nf) * (1.0 / (_N_PEGS - 1))

            rew_v[pl.ds(e0, 16)] = rew
            ndone_v[pl.ds(e0, 16)] = new_done
            npo_v[pl.ds(e0, 16)] = n_new
            trewo_v[pl.ds(e0, 16)] = trew_v[pl.ds(e0, 16)] + rew

            for i in range(7):
                for j in range(7):
                    bc = (i * 7 + j) * 3
                    ch0 = v[_POS_TO_IDX[(i, j)]] if (i, j) in _POS_TO_IDX else zeros
                    plsc.store_scatter(states_v, [rows, jnp.full((16,), bc, jnp.int32)], ch0)
                    plsc.store_scatter(states_v, [rows, jnp.full((16,), bc + 1, jnp.int32)], r1)
                    plsc.store_scatter(states_v, [rows, jnp.full((16,), bc + 2, jnp.int32)], r2)
            return gcarry

        lax.fori_loop(0, _NGROUP, group_body, 0)

        pltpu.sync_copy(states_v, states_hbm.at[pl.ds(base, _C)])
        pltpu.sync_copy(rew_v, rew_hbm.at[pl.ds(base, _C)])
        pltpu.sync_copy(ndone_v, ndone_hbm.at[pl.ds(base, _C)])
        pltpu.sync_copy(npo_v, npo_hbm.at[pl.ds(base, _C)])
        pltpu.sync_copy(trewo_v, trewo_hbm.at[pl.ds(base, _C)])
        return carry

    lax.fori_loop(0, _NCHUNK, chunk_body, 0)


_sc_call = pl.kernel(
    _sc_body,
    out_type=(
        jax.ShapeDtypeStruct((_N_ENVS, 147), jnp.float32),
        jax.ShapeDtypeStruct((_N_ENVS,), jnp.float32),
        jax.ShapeDtypeStruct((_N_ENVS,), jnp.int32),
        jax.ShapeDtypeStruct((_N_ENVS,), jnp.int32),
        jax.ShapeDtypeStruct((_N_ENVS,), jnp.float32),
    ),
    mesh=plsc.VectorSubcoreMesh(core_axis_name="c", subcore_axis_name="s"),
    compiler_params=pltpu.CompilerParams(needs_layout_passes=False),
    scratch_types=(
        pltpu.VMEM((_C, _N_POS), jnp.float32),
        pltpu.VMEM((_C, 147), jnp.float32),
        pltpu.VMEM((_C,), jnp.int32),
        pltpu.VMEM((_C,), jnp.int32),
        pltpu.VMEM((_C,), jnp.int32),
        pltpu.VMEM((_C,), jnp.float32),
        pltpu.VMEM((_C,), jnp.float32),
        pltpu.VMEM((_C,), jnp.int32),
        pltpu.VMEM((_C,), jnp.int32),
        pltpu.VMEM((_C,), jnp.float32),
        pltpu.VMEM((144,), jnp.int32),
        pltpu.VMEM((144,), jnp.int32),
    ),
)


@jax.jit
def kernel(pegs, total_reward, n_pegs, done, actions):
    n = pegs.shape[0]
    states_flat, rew, ndone, npo, trew_o = _sc_call(
        pegs, actions, n_pegs, done.astype(jnp.int32), total_reward,
        _AMID_PAD, _ATGT_PAD)
    states = states_flat.reshape(n, 7, 7, 3)
    return (rew, states, ndone.astype(jnp.bool_), npo, trew_o)
